# initial kernel scaffold (unmeasured)
import jax
import jax.numpy as jnp
from jax import lax
from jax.experimental import pallas as pl
from jax.experimental.pallas import tpu as pltpu


def kernel(
    x,
):
    def body(*refs):
        pass

    out_shape = jax.ShapeDtypeStruct(..., jnp.float32)
    return pl.pallas_call(body, out_shape=out_shape)(...)


# baseline (device time: 1602162 ns/iter reference)
import jax
import jax.numpy as jnp
from jax import lax
from jax.experimental import pallas as pl
from jax.experimental.pallas import tpu as pltpu

N_CHUNKS = 32


def kernel(x):
    m, n = x.shape
    chunk_m = m // N_CHUNKS

    def body(x_ref, out_ref, recv_ref, send_sems, recv_sems):
        my_x = lax.axis_index("x")
        my_y = lax.axis_index("y")
        my_z = lax.axis_index("z")
        peer = (my_x, 1 - my_y, my_z)
        step = pl.program_id(0)

        @pl.when(step == 0)
        def _():
            bar = pltpu.get_barrier_semaphore()
            pl.semaphore_signal(
                bar, inc=1, device_id=peer,
                device_id_type=pl.DeviceIdType.MESH,
            )
            pl.semaphore_wait(bar, 1)

        slot = lax.rem(step, 2)
        rdma = pltpu.make_async_remote_copy(
            src_ref=x_ref,
            dst_ref=recv_ref.at[slot],
            send_sem=send_sems.at[slot],
            recv_sem=recv_sems.at[slot],
            device_id=peer,
            device_id_type=pl.DeviceIdType.MESH,
        )
        rdma.start()
        rdma.wait()
        out_ref[...] = x_ref[...] + recv_ref[slot]

    return pl.pallas_call(
        body,
        grid=(N_CHUNKS,),
        in_specs=[pl.BlockSpec((chunk_m, n), lambda i: (i, 0))],
        out_specs=pl.BlockSpec((chunk_m, n), lambda i: (i, 0)),
        out_shape=jax.ShapeDtypeStruct((m, n), x.dtype),
        scratch_shapes=[
            pltpu.VMEM((2, chunk_m, n), x.dtype),
            pltpu.SemaphoreType.DMA((2,)),
            pltpu.SemaphoreType.DMA((2,)),
        ],
        compiler_params=pltpu.CompilerParams(
            collective_id=0,
            dimension_semantics=("arbitrary",),
            vmem_limit_bytes=64 * 1024 * 1024,
        ),
    )(x)


# device time: 918366 ns/iter; 1.7446x vs baseline; 1.7446x over previous
import jax
import jax.numpy as jnp
from jax import lax
from jax.experimental import pallas as pl
from jax.experimental.pallas import tpu as pltpu

N_PAIRS = 32
N_SLOTS = 4


def kernel(x):
    m, n = x.shape
    pair_m = m // N_PAIRS
    sub_m = pair_m // 2

    def body(x_any_ref, x_prev_ref, out_ref, recv_y, recv_x,
             y_send_sems, y_recv_sems, f_send_sems, f_recv_sems):
        my_x = lax.axis_index("x")
        my_y = lax.axis_index("y")
        my_z = lax.axis_index("z")
        ypeer = (my_x, 1 - my_y, my_z)
        xpeer = (1 - my_x, my_y, my_z)
        p = pl.program_id(0)

        def y_desc(q):
            slot = lax.rem(q, N_SLOTS)
            return pltpu.make_async_remote_copy(
                src_ref=x_any_ref.at[
                    pl.ds(q * pair_m + my_x * sub_m, sub_m), :],
                dst_ref=recv_y.at[slot],
                send_sem=y_send_sems.at[slot],
                recv_sem=y_recv_sems.at[slot],
                device_id=ypeer,
                device_id_type=pl.DeviceIdType.MESH,
            )

        def f_desc(q):
            slot = lax.rem(q, N_SLOTS)
            return pltpu.make_async_remote_copy(
                src_ref=recv_y.at[slot],
                dst_ref=recv_x.at[slot],
                send_sem=f_send_sems.at[slot],
                recv_sem=f_recv_sems.at[slot],
                device_id=xpeer,
                device_id_type=pl.DeviceIdType.MESH,
            )

        @pl.when(p == 0)
        def _():
            bar = pltpu.get_barrier_semaphore()
            for nbr in (ypeer, xpeer):
                pl.semaphore_signal(
                    bar, inc=1, device_id=nbr,
                    device_id_type=pl.DeviceIdType.MESH,
                )
            pl.semaphore_wait(bar, 2)

        @pl.when(p < N_PAIRS)
        def _():
            @pl.when(p >= N_SLOTS)
            def _():
                y_desc(p - N_SLOTS).wait_send()
            y_desc(p).start()

        @pl.when(p >= 1)
        def _():
            q = p - 1
            qslot = lax.rem(q, N_SLOTS)
            y_desc(q).wait_recv()
            @pl.when(q >= 1)
            def _():
                f_desc(q - 1).wait_send()
            f_desc(q).start()
            off_d = my_x * sub_m
            out_ref[pl.ds(off_d, sub_m), :] = (
                x_prev_ref[pl.ds(off_d, sub_m), :] + recv_y[qslot]
            )
            f_desc(q).wait_recv()
            off_f = (1 - my_x) * sub_m
            out_ref[pl.ds(off_f, sub_m), :] = (
                x_prev_ref[pl.ds(off_f, sub_m), :] + recv_x[qslot]
            )

        @pl.when(p == N_PAIRS)
        def _():
            for k in range(N_SLOTS):
                y_desc(N_PAIRS - N_SLOTS + k).wait_send()
            f_desc(N_PAIRS - 1).wait_send()

    grid = (N_PAIRS + 1,)
    return pl.pallas_call(
        body,
        grid=grid,
        in_specs=[
            pl.BlockSpec(memory_space=pltpu.MemorySpace.HBM),
            pl.BlockSpec((pair_m, n), lambda i: (jnp.maximum(i - 1, 0), 0)),
        ],
        out_specs=pl.BlockSpec(
            (pair_m, n), lambda i: (jnp.maximum(i - 1, 0), 0)
        ),
        out_shape=jax.ShapeDtypeStruct((m, n), x.dtype),
        scratch_shapes=[
            pltpu.VMEM((N_SLOTS, sub_m, n), x.dtype),
            pltpu.VMEM((N_SLOTS, sub_m, n), x.dtype),
            pltpu.SemaphoreType.DMA((N_SLOTS,)),
            pltpu.SemaphoreType.DMA((N_SLOTS,)),
            pltpu.SemaphoreType.DMA((N_SLOTS,)),
            pltpu.SemaphoreType.DMA((N_SLOTS,)),
        ],
        compiler_params=pltpu.CompilerParams(
            collective_id=0,
            dimension_semantics=("arbitrary",),
            vmem_limit_bytes=60 * 1024 * 1024,
        ),
    )(x, x)


# device time: 840100 ns/iter; 1.9071x vs baseline; 1.0932x over previous
import jax
import jax.numpy as jnp
from jax import lax
from jax.experimental import pallas as pl
from jax.experimental.pallas import tpu as pltpu

N_PAIRS = 32
N_SLOTS = 5


def kernel(x):
    m, n = x.shape
    pair_m = m // N_PAIRS
    sub_m = pair_m // 2

    def body(x_any_ref, x_prev_ref, out_ref, recv_y, recv_x,
             y_send_sems, y_recv_sems, f_send_sems, f_recv_sems):
        my_x = lax.axis_index("x")
        my_y = lax.axis_index("y")
        my_z = lax.axis_index("z")
        ypeer = (my_x, 1 - my_y, my_z)
        xpeer = (1 - my_x, my_y, my_z)
        p = pl.program_id(0)

        def y_desc(q):
            slot = lax.rem(q, N_SLOTS)
            return pltpu.make_async_remote_copy(
                src_ref=x_any_ref.at[
                    pl.ds(q * pair_m + my_x * sub_m, sub_m), :],
                dst_ref=recv_y.at[slot],
                send_sem=y_send_sems.at[slot],
                recv_sem=y_recv_sems.at[slot],
                device_id=ypeer,
                device_id_type=pl.DeviceIdType.MESH,
            )

        def f_desc(q):
            slot = lax.rem(q, N_SLOTS)
            return pltpu.make_async_remote_copy(
                src_ref=recv_y.at[slot],
                dst_ref=recv_x.at[slot],
                send_sem=f_send_sems.at[slot],
                recv_sem=f_recv_sems.at[slot],
                device_id=xpeer,
                device_id_type=pl.DeviceIdType.MESH,
            )

        @pl.when(p == 0)
        def _():
            bar = pltpu.get_barrier_semaphore()
            for nbr in (ypeer, xpeer):
                pl.semaphore_signal(
                    bar, inc=1, device_id=nbr,
                    device_id_type=pl.DeviceIdType.MESH,
                )
            pl.semaphore_wait(bar, 2)

        @pl.when(p >= 3)
        def _():
            f_desc(p - 3).wait_send()

        @pl.when(p < N_PAIRS)
        def _():
            @pl.when(p >= N_SLOTS)
            def _():
                y_desc(p - N_SLOTS).wait_send()
            y_desc(p).start()

        @pl.when((p >= 1) & (p <= N_PAIRS))
        def _():
            q = p - 1
            y_desc(q).wait_recv()
            f_desc(q).start()

        @pl.when(p >= 2)
        def _():
            r = p - 2
            rslot = lax.rem(r, N_SLOTS)
            off_d = my_x * sub_m
            out_ref[pl.ds(off_d, sub_m), :] = (
                x_prev_ref[pl.ds(off_d, sub_m), :] + recv_y[rslot]
            )
            f_desc(r).wait_recv()
            off_f = (1 - my_x) * sub_m
            out_ref[pl.ds(off_f, sub_m), :] = (
                x_prev_ref[pl.ds(off_f, sub_m), :] + recv_x[rslot]
            )

        @pl.when(p == N_PAIRS + 1)
        def _():
            f_desc(N_PAIRS - 1).wait_send()
            for k in range(N_SLOTS):
                y_desc(N_PAIRS - N_SLOTS + k).wait_send()

    grid = (N_PAIRS + 2,)
    return pl.pallas_call(
        body,
        grid=grid,
        in_specs=[
            pl.BlockSpec(memory_space=pltpu.MemorySpace.HBM),
            pl.BlockSpec((pair_m, n), lambda i: (jnp.maximum(i - 2, 0), 0)),
        ],
        out_specs=pl.BlockSpec(
            (pair_m, n), lambda i: (jnp.maximum(i - 2, 0), 0)
        ),
        out_shape=jax.ShapeDtypeStruct((m, n), x.dtype),
        scratch_shapes=[
            pltpu.VMEM((N_SLOTS, sub_m, n), x.dtype),
            pltpu.VMEM((N_SLOTS, sub_m, n), x.dtype),
            pltpu.SemaphoreType.DMA((N_SLOTS,)),
            pltpu.SemaphoreType.DMA((N_SLOTS,)),
            pltpu.SemaphoreType.DMA((N_SLOTS,)),
            pltpu.SemaphoreType.DMA((N_SLOTS,)),
        ],
        compiler_params=pltpu.CompilerParams(
            collective_id=0,
            dimension_semantics=("arbitrary",),
            vmem_limit_bytes=60 * 1024 * 1024,
        ),
    )(x, x)
